# Initial kernel scaffold; baseline (speedup 1.0000x reference)
import functools

import jax
import jax.numpy as jnp
from jax import lax
from jax.experimental import pallas as pl
from jax.experimental.pallas import tpu as pltpu
from jax.experimental.pallas import tpu_sc as plsc

NC = 2
NS = 16
NW = NC * NS
LN = 16
CH = 128

N_U = 50000
N_M = 10000
D = 64

R_U = 25088
R_M = 5120


def _zdiv(rows_pt):
    for z in range(min(rows_pt, 128), 0, -1):
        if rows_pt % z == 0:
            return z
    return 1


def _zero_rows(buf, nrows, width):
    z16 = jnp.zeros((LN,), jnp.float32)

    def body(i, c):
        for k in range(width // LN):
            buf[i, pl.ds(k * LN, LN)] = z16
        return c

    lax.fori_loop(0, nrows, body, 0)


def _fill_ones(buf, nrows):
    one16 = jnp.ones((LN,), jnp.float32)

    def body(i, c):
        buf[i, pl.ds(0, LN)] = one16
        return c

    lax.fori_loop(0, nrows, body, 0)


def _make_segsum(n_dst, r_shard, e_total):
    acc_rows = r_shard + LN
    rows_pt = r_shard // NS
    zr = _zdiv(rows_pt)
    nz = rows_pt // zr
    nchunks = e_total // CH
    npt = -(-nchunks // NS)

    mesh = plsc.VectorSubcoreMesh(core_axis_name="c", subcore_axis_name="s")

    @functools.partial(
        pl.kernel,
        out_type=jax.ShapeDtypeStruct((2 * r_shard, D), jnp.float32),
        mesh=mesh,
        scratch_types=[
            pltpu.VMEM((CH,), jnp.int32),
            pltpu.VMEM((CH,), jnp.int32),
            pltpu.VMEM((CH,), jnp.int32),
            pltpu.VMEM((CH, D), jnp.float32),
            pltpu.VMEM_SHARED((acc_rows, D), jnp.float32),
            pltpu.SemaphoreType.DMA,
        ],
    )
    def seg(src_hbm, dst_hbm, table_hbm, out_hbm, gbuf, draw, dloc, rows, acc, sem):
        c = lax.axis_index("c")
        s = lax.axis_index("s")
        lo = c * r_shard
        hi = jnp.minimum(n_dst, lo + r_shard)

        _zero_rows(rows, zr, D)

        def zbody(t, carry):
            pltpu.sync_copy(rows.at[pl.ds(0, zr)],
                            acc.at[pl.ds(s * rows_pt + t * zr, zr)])
            return carry

        lax.fori_loop(0, nz, zbody, 0)
        plsc.subcore_barrier()

        def chunk(jj, carry):
            j = jj * NS + s

            @pl.when(j < nchunks)
            def _():
                off = j * CH
                pltpu.sync_copy(src_hbm.at[pl.ds(off, CH)], gbuf)
                pltpu.sync_copy(dst_hbm.at[pl.ds(off, CH)], draw)
                for g in range(CH // LN):
                    d = draw[pl.ds(g * LN, LN)]
                    inr = (d >= lo) & (d < hi)
                    dloc[pl.ds(g * LN, LN)] = jnp.where(inr, d - lo, r_shard)
                pltpu.async_copy(table_hbm.at[gbuf], rows, sem).wait()
                pltpu.sync_copy(rows, acc.at[dloc], add=True)

            return carry

        lax.fori_loop(0, npt, chunk, 0)
        plsc.subcore_barrier()
        pltpu.sync_copy(acc.at[pl.ds(s * rows_pt, rows_pt)],
                        out_hbm.at[pl.ds(c * r_shard + s * rows_pt, rows_pt)])

    return seg


def _make_counts(e_total):
    rpt_u = R_U // NS
    rpt_m = R_M // NS
    zr_u = _zdiv(rpt_u)
    zr_m = _zdiv(rpt_m)
    nchunks = e_total // CH
    npt = -(-nchunks // NS)

    mesh = plsc.VectorSubcoreMesh(core_axis_name="c", subcore_axis_name="s")

    @functools.partial(
        pl.kernel,
        out_type=(jax.ShapeDtypeStruct((2 * R_U, LN), jnp.float32),
                  jax.ShapeDtypeStruct((2 * R_M, LN), jnp.float32)),
        mesh=mesh,
        scratch_types=[
            pltpu.VMEM((CH,), jnp.int32),
            pltpu.VMEM((CH,), jnp.int32),
            pltpu.VMEM((CH,), jnp.int32),
            pltpu.VMEM((CH,), jnp.int32),
            pltpu.VMEM((CH, LN), jnp.float32),
            pltpu.VMEM_SHARED((R_U + LN, LN), jnp.float32),
            pltpu.VMEM_SHARED((R_M + LN, LN), jnp.float32),
        ],
    )
    def cnts(eu_hbm, em_hbm, cu_hbm, cm_hbm, ubuf, mbuf, uloc, mloc, ones,
             accu, accm):
        c = lax.axis_index("c")
        s = lax.axis_index("s")
        lo_u = c * R_U
        hi_u = jnp.minimum(N_U, lo_u + R_U)
        lo_m = c * R_M
        hi_m = jnp.minimum(N_M, lo_m + R_M)

        _zero_rows(ones, max(zr_u, zr_m), LN)

        def zu(t, carry):
            pltpu.sync_copy(ones.at[pl.ds(0, zr_u)],
                            accu.at[pl.ds(s * rpt_u + t * zr_u, zr_u)])
            return carry

        lax.fori_loop(0, rpt_u // zr_u, zu, 0)

        def zm(t, carry):
            pltpu.sync_copy(ones.at[pl.ds(0, zr_m)],
                            accm.at[pl.ds(s * rpt_m + t * zr_m, zr_m)])
            return carry

        lax.fori_loop(0, rpt_m // zr_m, zm, 0)

        _fill_ones(ones, CH)
        plsc.subcore_barrier()

        def chunk(jj, carry):
            j = jj * NS + s

            @pl.when(j < nchunks)
            def _():
                off = j * CH
                pltpu.sync_copy(eu_hbm.at[pl.ds(off, CH)], ubuf)
                pltpu.sync_copy(em_hbm.at[pl.ds(off, CH)], mbuf)
                for g in range(CH // LN):
                    u = ubuf[pl.ds(g * LN, LN)]
                    m = mbuf[pl.ds(g * LN, LN)]
                    uin = (u >= lo_u) & (u < hi_u)
                    minr = (m >= lo_m) & (m < hi_m)
                    uloc[pl.ds(g * LN, LN)] = jnp.where(uin, u - lo_u, R_U)
                    mloc[pl.ds(g * LN, LN)] = jnp.where(minr, m - lo_m, R_M)
                pltpu.sync_copy(ones, accu.at[uloc], add=True)
                pltpu.sync_copy(ones, accm.at[mloc], add=True)

            return carry

        lax.fori_loop(0, npt, chunk, 0)
        plsc.subcore_barrier()
        pltpu.sync_copy(accu.at[pl.ds(s * rpt_u, rpt_u)],
                        cu_hbm.at[pl.ds(c * R_U + s * rpt_u, rpt_u)])
        pltpu.sync_copy(accm.at[pl.ds(s * rpt_m, rpt_m)],
                        cm_hbm.at[pl.ds(c * R_M + s * rpt_m, rpt_m)])

    return cnts


def _tc_layer(sum_pad, cnt16, x_pad, wl, b, wr, relu):
    rows = sum_pad.shape[0]
    br = 256
    grid = rows // br

    def body(sum_ref, cnt_ref, x_ref, wl_ref, b_ref, wr_ref, o_ref):
        cnt = jnp.sum(cnt_ref[...], axis=1, keepdims=True)
        inv = 1.0 / jnp.maximum(cnt, 1.0)
        mean = sum_ref[...] * inv
        out = (jnp.dot(mean, wl_ref[...], preferred_element_type=jnp.float32)
               + b_ref[...]
               + jnp.dot(x_ref[...], wr_ref[...],
                         preferred_element_type=jnp.float32))
        if relu:
            out = jnp.maximum(out, 0.0)
        o_ref[...] = out

    return pl.pallas_call(
        body,
        grid=(grid,),
        in_specs=[
            pl.BlockSpec((br, D), lambda i: (i, 0)),
            pl.BlockSpec((br, LN), lambda i: (i, 0)),
            pl.BlockSpec((br, D), lambda i: (i, 0)),
            pl.BlockSpec((D, D), lambda i: (0, 0)),
            pl.BlockSpec((1, D), lambda i: (0, 0)),
            pl.BlockSpec((D, D), lambda i: (0, 0)),
        ],
        out_specs=pl.BlockSpec((br, D), lambda i: (i, 0)),
        out_shape=jax.ShapeDtypeStruct((rows, D), jnp.float32),
    )(sum_pad, cnt16, x_pad, wl, b.reshape(1, D), wr)


def _make_decoder(b_pad):
    nchunks = b_pad // CH
    npt = -(-nchunks // NW)

    mesh = plsc.VectorSubcoreMesh(core_axis_name="c", subcore_axis_name="s")

    @functools.partial(
        pl.kernel,
        out_type=jax.ShapeDtypeStruct((b_pad,), jnp.float32),
        mesh=mesh,
        scratch_types=[
            pltpu.VMEM((CH,), jnp.int32),
            pltpu.VMEM((CH,), jnp.int32),
            pltpu.VMEM((CH, D), jnp.float32),
            pltpu.VMEM((CH, D), jnp.float32),
            pltpu.VMEM((LN, LN), jnp.float32),
            pltpu.VMEM((CH,), jnp.float32),
            pltpu.SemaphoreType.DMA,
            pltpu.SemaphoreType.DMA,
        ],
    )
    def dec(xu_hbm, xm_hbm, ls_hbm, ld_hbm, out_hbm,
            aidx, bidx, arows, brows, ptile, ochunk, sema, semb):
        c = lax.axis_index("c")
        s = lax.axis_index("s")
        w = c * NS + s
        iota16 = lax.iota(jnp.int32, LN)

        def chunk(k, carry):
            j = k * NW + w

            @pl.when(j < nchunks)
            def _():
                off = j * CH
                pltpu.sync_copy(ls_hbm.at[pl.ds(off, CH)], aidx)
                pltpu.sync_copy(ld_hbm.at[pl.ds(off, CH)], bidx)
                cpa = pltpu.async_copy(xu_hbm.at[aidx], arows, sema)
                cpb = pltpu.async_copy(xm_hbm.at[bidx], brows, semb)
                cpa.wait()
                cpb.wait()
                for g in range(CH // LN):
                    def rbody(r, carry2):
                        row = g * LN + r
                        acc = (arows[row, pl.ds(0, LN)]
                               * brows[row, pl.ds(0, LN)])
                        for q in range(1, D // LN):
                            acc = acc + (arows[row, pl.ds(q * LN, LN)]
                                         * brows[row, pl.ds(q * LN, LN)])
                        ptile[r, pl.ds(0, LN)] = acc
                        return carry2

                    lax.fori_loop(0, LN, rbody, 0)

                    def cbody(q, accv):
                        col = plsc.load_gather(
                            ptile, [iota16, jnp.full((LN,), q, jnp.int32)])
                        return accv + col

                    out16 = lax.fori_loop(0, LN, cbody,
                                          jnp.zeros((LN,), jnp.float32))
                    ochunk[pl.ds(g * LN, LN)] = out16
                pltpu.sync_copy(ochunk, out_hbm.at[pl.ds(off, CH)])

            return carry

        lax.fori_loop(0, npt, chunk, 0)

    return dec


def kernel(x_user, x_movie, params, edge_src_user, edge_dst_movie,
           label_src_user, label_dst_movie):
    eu = edge_src_user.astype(jnp.int32)
    em = edge_dst_movie.astype(jnp.int32)
    ls = label_src_user.astype(jnp.int32)
    ld = label_dst_movie.astype(jnp.int32)
    e_total = eu.shape[0]
    b_lab = ls.shape[0]
    b_pad = -(-b_lab // CH) * CH

    xu = jnp.pad(x_user, ((0, 2 * R_U - N_U), (0, 0)))
    xm = jnp.pad(x_movie, ((0, 2 * R_M - N_M), (0, 0)))
    ls_p = jnp.pad(ls, (0, b_pad - b_lab))
    ld_p = jnp.pad(ld, (0, b_pad - b_lab))

    counts = _make_counts(e_total)
    cnt_u16, cnt_m16 = counts(eu, em)

    seg_m = _make_segsum(N_M, R_M, e_total)
    seg_u = _make_segsum(N_U, R_U, e_total)

    p = params
    for l in range(3):
        sum_m = seg_m(eu, em, xu)
        sum_u = seg_u(em, eu, xm)
        relu = l < 2
        xm = _tc_layer(sum_m, cnt_m16, xm,
                       p["Wl%d_um" % l], p["bl%d_um" % l], p["Wr%d_um" % l],
                       relu)
        xu = _tc_layer(sum_u, cnt_u16, xu,
                       p["Wl%d_mu" % l], p["bl%d_mu" % l], p["Wr%d_mu" % l],
                       relu)

    dec = _make_decoder(b_pad)
    out = dec(xu, xm, ls_p, ld_p)
    return out[:b_lab]


# speedup vs baseline: 2.7589x; 2.7589x over previous
import functools

import jax
import jax.numpy as jnp
from jax import lax
from jax.experimental import pallas as pl
from jax.experimental.pallas import tpu as pltpu
from jax.experimental.pallas import tpu_sc as plsc

NC = 2
NS = 16
NW = NC * NS
LN = 16
CH = 128

N_U = 50000
N_M = 10000
D = 64

R_U = 25088
R_M = 5120


def _zdiv(rows_pt):
    for z in range(min(rows_pt, 128), 0, -1):
        if rows_pt % z == 0:
            return z
    return 1


def _zero_rows(buf, nrows, width):
    z16 = jnp.zeros((LN,), jnp.float32)

    def body(i, c):
        for k in range(width // LN):
            buf[i, pl.ds(k * LN, LN)] = z16
        return c

    lax.fori_loop(0, nrows, body, 0)


def _fill_ones(buf, nrows):
    one16 = jnp.ones((LN,), jnp.float32)

    def body(i, c):
        buf[i, pl.ds(0, LN)] = one16
        return c

    lax.fori_loop(0, nrows, body, 0)


def _make_segsum(n_dst, r_shard, e_total):
    acc_rows = r_shard + LN
    rows_pt = r_shard // NS
    zr = _zdiv(rows_pt)
    nz = rows_pt // zr
    nchunks = e_total // CH
    npt = -(-nchunks // NS)

    mesh = plsc.VectorSubcoreMesh(core_axis_name="c", subcore_axis_name="s")

    @functools.partial(
        pl.kernel,
        out_type=jax.ShapeDtypeStruct((2 * r_shard, D), jnp.float32),
        mesh=mesh,
        compiler_params=pltpu.CompilerParams(use_tc_tiling_on_sc=False),
        scratch_types=[
            pltpu.VMEM((CH,), jnp.int32),
            pltpu.VMEM((CH,), jnp.int32),
            pltpu.VMEM((CH,), jnp.int32),
            pltpu.VMEM((CH, D), jnp.float32),
            pltpu.VMEM_SHARED((acc_rows, D), jnp.float32),
            pltpu.SemaphoreType.DMA,
        ],
    )
    def seg(src_hbm, dst_hbm, table_hbm, out_hbm, gbuf, draw, dloc, rows, acc, sem):
        c = lax.axis_index("c")
        s = lax.axis_index("s")
        lo = c * r_shard
        hi = jnp.minimum(n_dst, lo + r_shard)

        _zero_rows(rows, zr, D)

        def zbody(t, carry):
            pltpu.sync_copy(rows.at[pl.ds(0, zr)],
                            acc.at[pl.ds(s * rows_pt + t * zr, zr)])
            return carry

        lax.fori_loop(0, nz, zbody, 0)
        plsc.subcore_barrier()

        def chunk(jj, carry):
            j = jj * NS + s

            @pl.when(j < nchunks)
            def _():
                off = j * CH
                pltpu.sync_copy(src_hbm.at[pl.ds(off, CH)], gbuf)
                pltpu.sync_copy(dst_hbm.at[pl.ds(off, CH)], draw)
                for g in range(CH // LN):
                    d = draw[pl.ds(g * LN, LN)]
                    inr = (d >= lo) & (d < hi)
                    dloc[pl.ds(g * LN, LN)] = jnp.where(inr, d - lo, r_shard)
                pltpu.async_copy(table_hbm.at[gbuf], rows, sem).wait()
                pltpu.sync_copy(rows, acc.at[dloc], add=True)

            return carry

        lax.fori_loop(0, npt, chunk, 0)
        plsc.subcore_barrier()
        pltpu.sync_copy(acc.at[pl.ds(s * rows_pt, rows_pt)],
                        out_hbm.at[pl.ds(c * r_shard + s * rows_pt, rows_pt)])

    return seg


def _make_counts(e_total):
    rpt_u = R_U // NS
    rpt_m = R_M // NS
    zr_u = _zdiv(rpt_u)
    zr_m = _zdiv(rpt_m)
    nchunks = e_total // CH
    npt = -(-nchunks // NS)

    mesh = plsc.VectorSubcoreMesh(core_axis_name="c", subcore_axis_name="s")

    @functools.partial(
        pl.kernel,
        out_type=(jax.ShapeDtypeStruct((2 * R_U, LN), jnp.float32),
                  jax.ShapeDtypeStruct((2 * R_M, LN), jnp.float32)),
        mesh=mesh,
        compiler_params=pltpu.CompilerParams(use_tc_tiling_on_sc=False),
        scratch_types=[
            pltpu.VMEM((CH,), jnp.int32),
            pltpu.VMEM((CH,), jnp.int32),
            pltpu.VMEM((CH,), jnp.int32),
            pltpu.VMEM((CH,), jnp.int32),
            pltpu.VMEM((CH, LN), jnp.float32),
            pltpu.VMEM_SHARED((R_U + LN, LN), jnp.float32),
            pltpu.VMEM_SHARED((R_M + LN, LN), jnp.float32),
        ],
    )
    def cnts(eu_hbm, em_hbm, cu_hbm, cm_hbm, ubuf, mbuf, uloc, mloc, ones,
             accu, accm):
        c = lax.axis_index("c")
        s = lax.axis_index("s")
        lo_u = c * R_U
        hi_u = jnp.minimum(N_U, lo_u + R_U)
        lo_m = c * R_M
        hi_m = jnp.minimum(N_M, lo_m + R_M)

        _zero_rows(ones, max(zr_u, zr_m), LN)

        def zu(t, carry):
            pltpu.sync_copy(ones.at[pl.ds(0, zr_u)],
                            accu.at[pl.ds(s * rpt_u + t * zr_u, zr_u)])
            return carry

        lax.fori_loop(0, rpt_u // zr_u, zu, 0)

        def zm(t, carry):
            pltpu.sync_copy(ones.at[pl.ds(0, zr_m)],
                            accm.at[pl.ds(s * rpt_m + t * zr_m, zr_m)])
            return carry

        lax.fori_loop(0, rpt_m // zr_m, zm, 0)

        _fill_ones(ones, CH)
        plsc.subcore_barrier()

        def chunk(jj, carry):
            j = jj * NS + s

            @pl.when(j < nchunks)
            def _():
                off = j * CH
                pltpu.sync_copy(eu_hbm.at[pl.ds(off, CH)], ubuf)
                pltpu.sync_copy(em_hbm.at[pl.ds(off, CH)], mbuf)
                for g in range(CH // LN):
                    u = ubuf[pl.ds(g * LN, LN)]
                    m = mbuf[pl.ds(g * LN, LN)]
                    uin = (u >= lo_u) & (u < hi_u)
                    minr = (m >= lo_m) & (m < hi_m)
                    uloc[pl.ds(g * LN, LN)] = jnp.where(uin, u - lo_u, R_U)
                    mloc[pl.ds(g * LN, LN)] = jnp.where(minr, m - lo_m, R_M)
                pltpu.sync_copy(ones, accu.at[uloc], add=True)
                pltpu.sync_copy(ones, accm.at[mloc], add=True)

            return carry

        lax.fori_loop(0, npt, chunk, 0)
        plsc.subcore_barrier()
        pltpu.sync_copy(accu.at[pl.ds(s * rpt_u, rpt_u)],
                        cu_hbm.at[pl.ds(c * R_U + s * rpt_u, rpt_u)])
        pltpu.sync_copy(accm.at[pl.ds(s * rpt_m, rpt_m)],
                        cm_hbm.at[pl.ds(c * R_M + s * rpt_m, rpt_m)])

    return cnts


def _tc_layer(sum_pad, cnt16, x_pad, wl, b, wr, relu):
    rows = sum_pad.shape[0]
    br = 256
    grid = rows // br

    def body(sum_ref, cnt_ref, x_ref, wl_ref, b_ref, wr_ref, o_ref):
        cnt = jnp.sum(cnt_ref[...], axis=1, keepdims=True) * (1.0 / LN)
        inv = 1.0 / jnp.maximum(cnt, 1.0)
        mean = sum_ref[...] * inv
        out = (jnp.dot(mean, wl_ref[...], preferred_element_type=jnp.float32)
               + b_ref[...]
               + jnp.dot(x_ref[...], wr_ref[...],
                         preferred_element_type=jnp.float32))
        if relu:
            out = jnp.maximum(out, 0.0)
        o_ref[...] = out

    return pl.pallas_call(
        body,
        grid=(grid,),
        in_specs=[
            pl.BlockSpec((br, D), lambda i: (i, 0)),
            pl.BlockSpec((br, LN), lambda i: (i, 0)),
            pl.BlockSpec((br, D), lambda i: (i, 0)),
            pl.BlockSpec((D, D), lambda i: (0, 0)),
            pl.BlockSpec((1, D), lambda i: (0, 0)),
            pl.BlockSpec((D, D), lambda i: (0, 0)),
        ],
        out_specs=pl.BlockSpec((br, D), lambda i: (i, 0)),
        out_shape=jax.ShapeDtypeStruct((rows, D), jnp.float32),
    )(sum_pad, cnt16, x_pad, wl, b.reshape(1, D), wr)


def _make_decoder(b_pad):
    nchunks = b_pad // CH
    npt = -(-nchunks // NW)

    mesh = plsc.VectorSubcoreMesh(core_axis_name="c", subcore_axis_name="s")

    @functools.partial(
        pl.kernel,
        out_type=jax.ShapeDtypeStruct((b_pad,), jnp.float32),
        mesh=mesh,
        compiler_params=pltpu.CompilerParams(use_tc_tiling_on_sc=False,
                                             needs_layout_passes=False),
        scratch_types=[
            pltpu.VMEM((CH,), jnp.int32),
            pltpu.VMEM((CH,), jnp.int32),
            pltpu.VMEM((CH, D), jnp.float32),
            pltpu.VMEM((CH, D), jnp.float32),
            pltpu.VMEM((LN, LN), jnp.float32),
            pltpu.VMEM((CH,), jnp.float32),
            pltpu.SemaphoreType.DMA,
            pltpu.SemaphoreType.DMA,
        ],
    )
    def dec(xu_hbm, xm_hbm, ls_hbm, ld_hbm, out_hbm,
            aidx, bidx, arows, brows, ptile, ochunk, sema, semb):
        c = lax.axis_index("c")
        s = lax.axis_index("s")
        w = c * NS + s
        iota16 = lax.iota(jnp.int32, LN)

        def chunk(k, carry):
            j = k * NW + w

            @pl.when(j < nchunks)
            def _():
                off = j * CH
                pltpu.sync_copy(ls_hbm.at[pl.ds(off, CH)], aidx)
                pltpu.sync_copy(ld_hbm.at[pl.ds(off, CH)], bidx)
                cpa = pltpu.async_copy(xu_hbm.at[aidx], arows, sema)
                cpb = pltpu.async_copy(xm_hbm.at[bidx], brows, semb)
                cpa.wait()
                cpb.wait()
                for g in range(CH // LN):
                    def rbody(r, carry2):
                        row = g * LN + r
                        acc = (arows[row, pl.ds(0, LN)]
                               * brows[row, pl.ds(0, LN)])
                        for q in range(1, D // LN):
                            acc = acc + (arows[row, pl.ds(q * LN, LN)]
                                         * brows[row, pl.ds(q * LN, LN)])
                        ptile[r, pl.ds(0, LN)] = acc
                        return carry2

                    lax.fori_loop(0, LN, rbody, 0)

                    def cbody(q, accv):
                        col = plsc.load_gather(
                            ptile, [iota16, jnp.full((LN,), q, jnp.int32)])
                        return accv + col

                    out16 = lax.fori_loop(0, LN, cbody,
                                          jnp.zeros((LN,), jnp.float32))
                    ochunk[pl.ds(g * LN, LN)] = out16
                pltpu.sync_copy(ochunk, out_hbm.at[pl.ds(off, CH)])

            return carry

        lax.fori_loop(0, npt, chunk, 0)

    return dec


def kernel(x_user, x_movie, params, edge_src_user, edge_dst_movie,
           label_src_user, label_dst_movie):
    eu = edge_src_user.astype(jnp.int32)
    em = edge_dst_movie.astype(jnp.int32)
    ls = label_src_user.astype(jnp.int32)
    ld = label_dst_movie.astype(jnp.int32)
    e_total = eu.shape[0]
    b_lab = ls.shape[0]
    b_pad = -(-b_lab // CH) * CH

    xu = jnp.pad(x_user, ((0, 2 * R_U - N_U), (0, 0)))
    xm = jnp.pad(x_movie, ((0, 2 * R_M - N_M), (0, 0)))
    ls_p = jnp.pad(ls, (0, b_pad - b_lab))
    ld_p = jnp.pad(ld, (0, b_pad - b_lab))

    counts = _make_counts(e_total)
    cnt_u16, cnt_m16 = counts(eu, em)

    seg_m = _make_segsum(N_M, R_M, e_total)
    seg_u = _make_segsum(N_U, R_U, e_total)

    p = params
    for l in range(3):
        sum_m = seg_m(eu, em, xu)
        sum_u = seg_u(em, eu, xm)
        relu = l < 2
        xm = _tc_layer(sum_m, cnt_m16, xm,
                       p["Wl%d_um" % l], p["bl%d_um" % l], p["Wr%d_um" % l],
                       relu)
        xu = _tc_layer(sum_u, cnt_u16, xu,
                       p["Wl%d_mu" % l], p["bl%d_mu" % l], p["Wr%d_mu" % l],
                       relu)

    dec = _make_decoder(b_pad)
    out = dec(xu, xm, ls_p, ld_p)
    return out[:b_lab]
